# baseline (speedup 1.0000x reference)
import functools

import jax
import jax.numpy as jnp
from jax import lax
from jax.experimental import pallas as pl
from jax.experimental.pallas import tpu as pltpu
from jax.experimental.pallas import tpu_sc as plsc

S = 8192
N_IMG = 4096
D = 4096
K = 1024

_NC = 2
_NS = 16
_NW = _NC * _NS
_RPW_G = K // _NW
_RPW_C = (S - N_IMG) // _NW
_GCH = 8
_NV = N_IMG // 16


_MB = 8
_MROWS = 128


def _matsel_body(q_ref, f_hbm, dk_ref, fc_ref, *scratch):
    bufs = scratch[:_MB]
    sems = scratch[_MB:]
    q = q_ref[...]
    n_chunks = N_IMG // _MROWS

    for b in range(_MB):
        pltpu.async_copy(
            f_hbm.at[pl.ds(b * _MROWS, _MROWS)], bufs[b], sems[b])

    ri = lax.broadcasted_iota(jnp.int32, (32, 128), 0)

    def chunk_loop(g, acc):
        for b in range(_MB):
            c = g * _MB + b
            pltpu.make_async_copy(
                f_hbm.at[pl.ds(c * _MROWS, _MROWS)], bufs[b], sems[b]
            ).wait()
            s = lax.dot_general(
                q, bufs[b][...], (((1,), (1,)), ((), ())),
                preferred_element_type=jnp.float32)

            @pl.when(c + _MB < n_chunks)
            def _():
                pltpu.async_copy(
                    f_hbm.at[pl.ds((c + _MB) * _MROWS, _MROWS)],
                    bufs[b], sems[b])

            acc = jnp.where(ri == c, jnp.broadcast_to(s, (32, 128)), acc)
        return acc

    scores = lax.fori_loop(
        0, n_chunks // _MB, chunk_loop, jnp.zeros((32, 128), jnp.float32))
    _select_from_scores(scores, dk_ref, fc_ref)


def _matsel_call(q2, feat):
    return pl.pallas_call(
        _matsel_body,
        in_specs=[
            pl.BlockSpec((1, D), memory_space=pltpu.MemorySpace.VMEM),
            pl.BlockSpec(memory_space=pl.ANY),
        ],
        out_specs=[
            pl.BlockSpec((32, 128), memory_space=pltpu.MemorySpace.VMEM),
            pl.BlockSpec((32, 128), memory_space=pltpu.MemorySpace.VMEM),
        ],
        out_shape=[
            jax.ShapeDtypeStruct((32, 128), jnp.int32),
            jax.ShapeDtypeStruct((32, 128), jnp.int32),
        ],
        scratch_shapes=[pltpu.VMEM((_MROWS, D), jnp.float32)] * _MB
        + [pltpu.SemaphoreType.DMA] * _MB,
    )(q2, feat)


def _select_from_scores(s, dk_ref, fc_ref):
    b = lax.bitcast_convert_type(s, jnp.int32)
    key = jnp.where(b >= 0, b, b ^ jnp.int32(0x7FFFFFFF))

    def bs(_, lh):
        lo, hi = lh
        fl = (lo >> 1) + (hi >> 1) + (lo & hi & jnp.int32(1))
        mid = fl + ((lo ^ hi) & jnp.int32(1))
        cnt = jnp.sum((key >= mid).astype(jnp.int32))
        big = cnt >= K
        return (jnp.where(big, mid, lo), jnp.where(big, hi, mid - 1))

    lo, _ = lax.fori_loop(
        0, 33, bs, (jnp.int32(-2147483648), jnp.int32(2147483647)))
    t = lo
    gt = key > t
    eq = key == t
    need_eq = (K - jnp.sum(gt.astype(jnp.int32))).astype(jnp.float32)

    ia = lax.broadcasted_iota(jnp.int32, (128, 128), 0)
    ib = lax.broadcasted_iota(jnp.int32, (128, 128), 1)
    um = (ia <= ib).astype(jnp.float32)
    ra = lax.broadcasted_iota(jnp.int32, (32, 32), 0)
    rb = lax.broadcasted_iota(jnp.int32, (32, 32), 1)
    lm = (rb < ra).astype(jnp.float32)

    def excl_cumsum(xf):
        incl_row = lax.dot_general(
            xf, um, (((1,), (0,)), ((), ())),
            preferred_element_type=jnp.float32)
        off = jnp.sum(
            lax.dot_general(lm, xf, (((1,), (0,)), ((), ())),
                            preferred_element_type=jnp.float32),
            axis=1, keepdims=True)
        return incl_row + off - xf

    eq_rank = excl_cumsum(eq.astype(jnp.float32))
    keep = gt | (eq & (eq_rank < need_eq))
    dest = excl_cumsum(keep.astype(jnp.float32))
    desti = dest.astype(jnp.int32)
    dk_ref[...] = jnp.where(keep, desti, jnp.int32(-1))
    fc_ref[...] = desti + keep.astype(jnp.int32)


_CNB = 4
_CCH = 4
_NCC = _RPW_C // _CCH
_NG = _RPW_G // _GCH


def _sc_copy_body(emb_hbm, out_hbm, c0_v, c1_v, c2_v, c3_v,
                  si0, si1, si2, si3, so0, so1, so2, so3):
    wid = lax.axis_index("s") * _NC + lax.axis_index("c")
    csrc = N_IMG + wid * _RPW_C
    cdst = K + wid * _RPW_C
    cbufs = (c0_v, c1_v, c2_v, c3_v)
    isems = (si0, si1, si2, si3)
    osems = (so0, so1, so2, so3)
    for b in range(_CNB):
        pltpu.async_copy(
            emb_hbm.at[pl.ds(csrc + b * _CCH, _CCH)], cbufs[b], isems[b])

    def copy_body(g, carry):
        for b in range(_CNB):
            c = g * _CNB + b
            pltpu.make_async_copy(
                emb_hbm.at[pl.ds(csrc + c * _CCH, _CCH)], cbufs[b], isems[b]
            ).wait()
            pltpu.async_copy(
                cbufs[b], out_hbm.at[pl.ds(cdst + c * _CCH, _CCH)], osems[b])
            b2 = (b + 2) % _CNB

            @pl.when(c >= 2)
            def _():
                pltpu.make_async_copy(
                    cbufs[b2], out_hbm.at[pl.ds(cdst, _CCH)], osems[b2]
                ).wait()

                @pl.when(c + 2 < _NCC)
                def _():
                    pltpu.async_copy(
                        emb_hbm.at[pl.ds(csrc + (c + 2) * _CCH, _CCH)],
                        cbufs[b2], isems[b2])
        return carry

    lax.fori_loop(0, _NCC // _CNB, copy_body, jnp.int32(0))
    for b in ((_NCC - 2) % _CNB, (_NCC - 1) % _CNB):
        pltpu.make_async_copy(
            cbufs[b], out_hbm.at[pl.ds(cdst, _CCH)], osems[b]).wait()


def _sc_gather_body(feat_hbm, dk_hbm, fc_hbm, out_hbm, dk_v, fc_v, idx_v,
                    g0_v, g1_v, sem0, sem1):
    wid = lax.axis_index("s") * _NC + lax.axis_index("c")
    pltpu.sync_copy(dk_hbm, dk_v)
    pltpu.sync_copy(fc_hbm, fc_v)
    lo = wid * _RPW_G
    hi = lo + _RPW_G

    def _vreg(ref, v):
        return ref[v // 8, pl.ds((v % 8) * 16, 16)]

    def lower_bound(thresh, last_lane):
        def bs(_, lh):
            l, h = lh
            mid = (l + h) // 2
            f = _vreg(fc_v, jnp.minimum(mid, _NV - 1))
            probe = jnp.max(f) if last_lane else jnp.min(f)
            good = probe >= thresh
            active = l < h
            return (jnp.where(active & ~good, mid + 1, l),
                    jnp.where(active & good, mid, h))

        l, _ = lax.fori_loop(0, 9, bs, (jnp.int32(0), jnp.int32(_NV)))
        return l

    v_start = lower_bound(lo + 1, True)
    v_end = lower_bound(hi + 1, False)

    def scan_body(v, carry):
        d = _vreg(dk_v, v)
        m = (d >= lo) & (d < hi)
        ivals = lax.iota(jnp.int32, 16) + v * 16
        plsc.store_scatter(idx_v, [jnp.where(m, d - lo, 0)], ivals, mask=m)
        return carry

    lax.fori_loop(v_start, v_end, scan_body, jnp.int32(0))

    gbufs = (g0_v, g1_v)
    gsems = (sem0, sem1)
    for b in range(2):
        pltpu.async_copy(
            feat_hbm.at[idx_v.at[pl.ds(b * _GCH, _GCH)]], gbufs[b], gsems[b])

    def gather_body(g, carry):
        for b in range(2):
            c = g * 2 + b
            pltpu.make_async_copy(
                feat_hbm.at[idx_v.at[pl.ds(c * _GCH, _GCH)]], gbufs[b],
                gsems[b]).wait()
            pltpu.sync_copy(
                gbufs[b], out_hbm.at[pl.ds(wid * _RPW_G + c * _GCH, _GCH)])

            @pl.when(c + 2 < _NG)
            def _():
                pltpu.async_copy(
                    feat_hbm.at[idx_v.at[pl.ds((c + 2) * _GCH, _GCH)]],
                    gbufs[b], gsems[b])
        return carry

    lax.fori_loop(0, _NG // 2, gather_body, jnp.int32(0))


@functools.cache
def _sc_copy():
    return pl.kernel(
        _sc_copy_body,
        out_type=jax.ShapeDtypeStruct((K + S - N_IMG, D), jnp.float32),
        mesh=plsc.VectorSubcoreMesh(core_axis_name="c", subcore_axis_name="s"),
        scratch_types=[pltpu.VMEM((_CCH, D), jnp.float32)] * _CNB
        + [pltpu.SemaphoreType.DMA] * (2 * _CNB),
        compiler_params=pltpu.CompilerParams(needs_layout_passes=False),
    )


@functools.cache
def _sc_gather():
    return pl.kernel(
        _sc_gather_body,
        out_type=(),
        mesh=plsc.VectorSubcoreMesh(core_axis_name="c", subcore_axis_name="s"),
        scratch_types=[
            pltpu.VMEM((32, 128), jnp.int32),
            pltpu.VMEM((32, 128), jnp.int32),
            pltpu.VMEM((_RPW_G,), jnp.int32),
            pltpu.VMEM((_GCH, D), jnp.float32),
            pltpu.VMEM((_GCH, D), jnp.float32),
            pltpu.SemaphoreType.DMA,
            pltpu.SemaphoreType.DMA,
        ],
        compiler_params=pltpu.CompilerParams(needs_layout_passes=False),
    )


@jax.jit
def kernel(input_ids, inputs_embeds, image_features, question_embed):
    del input_ids
    out = _sc_copy()(inputs_embeds.reshape(S, D))
    dkey, fcum = _matsel_call(question_embed.reshape(1, D), image_features)
    out_ref = jax.new_ref(out)
    _sc_gather()(image_features, dkey, fcum, out_ref)
    return jax.freeze(out_ref).reshape(1, K + S - N_IMG, D)


# speedup vs baseline: 1.0474x; 1.0474x over previous
import functools

import jax
import jax.numpy as jnp
from jax import lax
from jax.experimental import pallas as pl
from jax.experimental.pallas import tpu as pltpu
from jax.experimental.pallas import tpu_sc as plsc

S = 8192
N_IMG = 4096
D = 4096
K = 1024

_NC = 2
_NS = 16
_NW = _NC * _NS
_RPW_G = K // _NW
_RPW_C = (S - N_IMG) // _NW
_GCH = 8
_NV = N_IMG // 16


_MB = 8
_MROWS = 128


def _matsel_body(q_ref, f_hbm, dk_ref, fc_ref, *scratch):
    bufs = scratch[:_MB]
    sems = scratch[_MB:]
    q = q_ref[...]
    n_chunks = N_IMG // _MROWS

    for b in range(_MB):
        pltpu.async_copy(
            f_hbm.at[pl.ds(b * _MROWS, _MROWS)], bufs[b], sems[b])

    ri = lax.broadcasted_iota(jnp.int32, (32, 128), 0)

    def chunk_loop(g, acc):
        for b in range(_MB):
            c = g * _MB + b
            pltpu.make_async_copy(
                f_hbm.at[pl.ds(c * _MROWS, _MROWS)], bufs[b], sems[b]
            ).wait()
            s = lax.dot_general(
                q, bufs[b][...], (((1,), (1,)), ((), ())),
                preferred_element_type=jnp.float32)

            @pl.when(c + _MB < n_chunks)
            def _():
                pltpu.async_copy(
                    f_hbm.at[pl.ds((c + _MB) * _MROWS, _MROWS)],
                    bufs[b], sems[b])

            acc = jnp.where(ri == c, jnp.broadcast_to(s, (32, 128)), acc)
        return acc

    scores = lax.fori_loop(
        0, n_chunks // _MB, chunk_loop, jnp.zeros((32, 128), jnp.float32))
    _select_from_scores(scores, dk_ref, fc_ref)


def _matsel_call(q2, feat):
    return pl.pallas_call(
        _matsel_body,
        in_specs=[
            pl.BlockSpec((1, D), memory_space=pltpu.MemorySpace.VMEM),
            pl.BlockSpec(memory_space=pl.ANY),
        ],
        out_specs=[
            pl.BlockSpec((32, 128), memory_space=pltpu.MemorySpace.VMEM),
            pl.BlockSpec((32, 128), memory_space=pltpu.MemorySpace.VMEM),
        ],
        out_shape=[
            jax.ShapeDtypeStruct((32, 128), jnp.int32),
            jax.ShapeDtypeStruct((32, 128), jnp.int32),
        ],
        scratch_shapes=[pltpu.VMEM((_MROWS, D), jnp.float32)] * _MB
        + [pltpu.SemaphoreType.DMA] * _MB,
    )(q2, feat)


def _select_from_scores(s, dk_ref, fc_ref):
    b = lax.bitcast_convert_type(s, jnp.int32)
    key = jnp.where(b >= 0, b, b ^ jnp.int32(0x7FFFFFFF))

    def bs(_, lh):
        lo, hi = lh
        fl = (lo >> 1) + (hi >> 1) + (lo & hi & jnp.int32(1))
        mid = fl + ((lo ^ hi) & jnp.int32(1))
        cnt = jnp.sum((key >= mid).astype(jnp.int32))
        big = cnt >= K
        return (jnp.where(big, mid, lo), jnp.where(big, hi, mid - 1))

    lo, _ = lax.fori_loop(
        0, 33, bs, (jnp.int32(-2147483648), jnp.int32(2147483647)))
    t = lo
    gt = key > t
    eq = key == t
    need_eq = (K - jnp.sum(gt.astype(jnp.int32))).astype(jnp.float32)

    ia = lax.broadcasted_iota(jnp.int32, (128, 128), 0)
    ib = lax.broadcasted_iota(jnp.int32, (128, 128), 1)
    um = (ia <= ib).astype(jnp.float32)
    ra = lax.broadcasted_iota(jnp.int32, (32, 32), 0)
    rb = lax.broadcasted_iota(jnp.int32, (32, 32), 1)
    lm = (rb < ra).astype(jnp.float32)

    def excl_cumsum(xf):
        incl_row = lax.dot_general(
            xf, um, (((1,), (0,)), ((), ())),
            preferred_element_type=jnp.float32)
        off = jnp.sum(
            lax.dot_general(lm, xf, (((1,), (0,)), ((), ())),
                            preferred_element_type=jnp.float32),
            axis=1, keepdims=True)
        return incl_row + off - xf

    eq_rank = excl_cumsum(eq.astype(jnp.float32))
    keep = gt | (eq & (eq_rank < need_eq))
    dest = excl_cumsum(keep.astype(jnp.float32))
    desti = dest.astype(jnp.int32)
    dk_ref[...] = jnp.where(keep, desti, jnp.int32(-1))
    fc_ref[...] = desti + keep.astype(jnp.int32)


_CNB = 4
_CCH = 4
_NCC = _RPW_C // _CCH
_NG = _RPW_G // _GCH


def _sc_copy_body(emb_hbm, out_hbm, sp, si0, si1, si2, si3,
                  so0, so1, so2, so3):
    wid = lax.axis_index("s") * _NC + lax.axis_index("c")
    sid = lax.axis_index("s")
    csrc = N_IMG + wid * _RPW_C
    cdst = K + wid * _RPW_C
    base = sid * (_CNB * _CCH)
    cbufs = tuple(
        sp.at[pl.ds(base + b * _CCH, _CCH)] for b in range(_CNB))
    isems = (si0, si1, si2, si3)
    osems = (so0, so1, so2, so3)
    for b in range(_CNB):
        pltpu.async_copy(
            emb_hbm.at[pl.ds(csrc + b * _CCH, _CCH)], cbufs[b], isems[b])

    def copy_body(g, carry):
        for b in range(_CNB):
            c = g * _CNB + b
            pltpu.make_async_copy(
                emb_hbm.at[pl.ds(csrc + c * _CCH, _CCH)], cbufs[b], isems[b]
            ).wait()
            pltpu.async_copy(
                cbufs[b], out_hbm.at[pl.ds(cdst + c * _CCH, _CCH)], osems[b])
            b2 = (b + 2) % _CNB

            @pl.when(c >= 2)
            def _():
                pltpu.make_async_copy(
                    cbufs[b2], out_hbm.at[pl.ds(cdst, _CCH)], osems[b2]
                ).wait()

                @pl.when(c + 2 < _NCC)
                def _():
                    pltpu.async_copy(
                        emb_hbm.at[pl.ds(csrc + (c + 2) * _CCH, _CCH)],
                        cbufs[b2], isems[b2])
        return carry

    lax.fori_loop(0, _NCC // _CNB, copy_body, jnp.int32(0))
    for b in ((_NCC - 2) % _CNB, (_NCC - 1) % _CNB):
        pltpu.make_async_copy(
            cbufs[b], out_hbm.at[pl.ds(cdst, _CCH)], osems[b]).wait()


def _sc_gather_body(feat_hbm, dk_hbm, fc_hbm, out_hbm, dk_v, fc_v, idx_v,
                    g0_v, g1_v, sem0, sem1):
    wid = lax.axis_index("s") * _NC + lax.axis_index("c")
    pltpu.sync_copy(dk_hbm, dk_v)
    pltpu.sync_copy(fc_hbm, fc_v)
    lo = wid * _RPW_G
    hi = lo + _RPW_G

    def _vreg(ref, v):
        return ref[v // 8, pl.ds((v % 8) * 16, 16)]

    def lower_bound(thresh, last_lane):
        def bs(_, lh):
            l, h = lh
            mid = (l + h) // 2
            f = _vreg(fc_v, jnp.minimum(mid, _NV - 1))
            probe = jnp.max(f) if last_lane else jnp.min(f)
            good = probe >= thresh
            active = l < h
            return (jnp.where(active & ~good, mid + 1, l),
                    jnp.where(active & good, mid, h))

        l, _ = lax.fori_loop(0, 9, bs, (jnp.int32(0), jnp.int32(_NV)))
        return l

    v_start = lower_bound(lo + 1, True)
    v_end = lower_bound(hi + 1, False)

    def scan_body(v, carry):
        d = _vreg(dk_v, v)
        m = (d >= lo) & (d < hi)
        ivals = lax.iota(jnp.int32, 16) + v * 16
        plsc.store_scatter(idx_v, [jnp.where(m, d - lo, 0)], ivals, mask=m)
        return carry

    lax.fori_loop(v_start, v_end, scan_body, jnp.int32(0))

    gbufs = (g0_v, g1_v)
    gsems = (sem0, sem1)
    for b in range(2):
        pltpu.async_copy(
            feat_hbm.at[idx_v.at[pl.ds(b * _GCH, _GCH)]], gbufs[b], gsems[b])

    def gather_body(g, carry):
        for b in range(2):
            c = g * 2 + b
            pltpu.make_async_copy(
                feat_hbm.at[idx_v.at[pl.ds(c * _GCH, _GCH)]], gbufs[b],
                gsems[b]).wait()
            pltpu.sync_copy(
                gbufs[b], out_hbm.at[pl.ds(wid * _RPW_G + c * _GCH, _GCH)])

            @pl.when(c + 2 < _NG)
            def _():
                pltpu.async_copy(
                    feat_hbm.at[idx_v.at[pl.ds((c + 2) * _GCH, _GCH)]],
                    gbufs[b], gsems[b])
        return carry

    lax.fori_loop(0, _NG // 2, gather_body, jnp.int32(0))


@functools.cache
def _sc_copy():
    return pl.kernel(
        _sc_copy_body,
        out_type=jax.ShapeDtypeStruct((K + S - N_IMG, D), jnp.float32),
        mesh=plsc.VectorSubcoreMesh(core_axis_name="c", subcore_axis_name="s"),
        scratch_types=[pltpu.VMEM_SHARED((_NS * _CNB * _CCH, D), jnp.float32)]
        + [pltpu.SemaphoreType.DMA] * (2 * _CNB),
        compiler_params=pltpu.CompilerParams(needs_layout_passes=False),
    )


@functools.cache
def _sc_gather():
    return pl.kernel(
        _sc_gather_body,
        out_type=(),
        mesh=plsc.VectorSubcoreMesh(core_axis_name="c", subcore_axis_name="s"),
        scratch_types=[
            pltpu.VMEM((32, 128), jnp.int32),
            pltpu.VMEM((32, 128), jnp.int32),
            pltpu.VMEM((_RPW_G,), jnp.int32),
            pltpu.VMEM((_GCH, D), jnp.float32),
            pltpu.VMEM((_GCH, D), jnp.float32),
            pltpu.SemaphoreType.DMA,
            pltpu.SemaphoreType.DMA,
        ],
        compiler_params=pltpu.CompilerParams(needs_layout_passes=False),
    )


@jax.jit
def kernel(input_ids, inputs_embeds, image_features, question_embed):
    del input_ids
    out = _sc_copy()(inputs_embeds.reshape(S, D))
    dkey, fcum = _matsel_call(question_embed.reshape(1, D), image_features)
    out_ref = jax.new_ref(out)
    _sc_gather()(image_features, dkey, fcum, out_ref)
    return jax.freeze(out_ref).reshape(1, K + S - N_IMG, D)
